# Initial kernel scaffold; baseline (speedup 1.0000x reference)
import jax
import jax.numpy as jnp
from jax.experimental import pallas as pl


def kernel(x, edge_index, ptr, W1, b1, W2, b2):
    raise NotImplementedError("write your pallas kernel here")


# speedup vs baseline: 12.2831x; 12.2831x over previous
import functools

import jax
import jax.numpy as jnp
from jax import lax
from jax.experimental import pallas as pl
from jax.experimental.pallas import tpu as pltpu
from jax.experimental.pallas import tpu_sc as plsc

N = 10000
E = 320000
D = 128
B = 256

NC = 2
NS = 16
NW = NC * NS

EBLK = 128
NB = (E + NW * EBLK - 1) // (NW * EBLK)
E_PAD = NW * NB * EBLK
ROWS_PT = 640
NR = NS * ROWS_PT

_mesh = plsc.VectorSubcoreMesh(core_axis_name="c", subcore_axis_name="s")


def _zero_fill(zbuf, nrows, ncols):
    z16 = jnp.zeros((16,), jnp.float32)
    for r in range(nrows):
        for c in range(ncols // 16):
            zbuf[r, pl.ds(c * 16, 16)] = z16


@functools.partial(
    pl.kernel, mesh=_mesh,
    out_type=jax.ShapeDtypeStruct((NC, NR, 16), jnp.float32),
    scratch_types=[
        pltpu.VMEM((NB, EBLK), jnp.int32),
        pltpu.VMEM((EBLK, 16), jnp.float32),
        pltpu.VMEM((16, 16), jnp.float32),
        pltpu.VMEM_SHARED((NR, 16), jnp.float32),
    ],
)
def _sc_deg(dst_hbm, out_hbm, dst_v, ones_v, zbuf, deg_sh):
    c = lax.axis_index("c")
    s = lax.axis_index("s")
    wid = c * NS + s
    pltpu.sync_copy(dst_hbm.at[wid], dst_v)
    one16 = jnp.ones((16,), jnp.float32)
    for r in range(EBLK):
        ones_v[r] = one16
    _zero_fill(zbuf, 16, 16)
    base = s * ROWS_PT
    for b in range(ROWS_PT // 16):
        pltpu.sync_copy(zbuf, deg_sh.at[pl.ds(base + b * 16, 16)])
    plsc.subcore_barrier()

    def body(j, carry):
        pltpu.sync_copy(ones_v, deg_sh.at[dst_v.at[j]], add=True)
        return carry

    lax.fori_loop(0, NB, body, 0)
    plsc.subcore_barrier()
    pltpu.sync_copy(deg_sh.at[pl.ds(base, ROWS_PT)],
                    out_hbm.at[c, pl.ds(base, ROWS_PT)])


@functools.partial(
    pl.kernel, mesh=_mesh,
    out_type=jax.ShapeDtypeStruct((NC, NR, D), jnp.float32),
    scratch_types=[
        pltpu.VMEM((NB, EBLK), jnp.int32),
        pltpu.VMEM((NB, EBLK), jnp.int32),
        pltpu.VMEM((EBLK, D), jnp.float32),
        pltpu.VMEM((16, D), jnp.float32),
        pltpu.VMEM_SHARED((NR, D), jnp.float32),
    ],
)
def _sc_edge(xs_hbm, src_hbm, dst_hbm, out_hbm, src_v, dst_v, rows_v, zbuf, acc_sh):
    c = lax.axis_index("c")
    s = lax.axis_index("s")
    wid = c * NS + s
    pltpu.sync_copy(src_hbm.at[wid], src_v)
    pltpu.sync_copy(dst_hbm.at[wid], dst_v)
    _zero_fill(zbuf, 16, D)
    base = s * ROWS_PT
    for b in range(ROWS_PT // 16):
        pltpu.sync_copy(zbuf, acc_sh.at[pl.ds(base + b * 16, 16)])
    plsc.subcore_barrier()

    def body(j, carry):
        pltpu.sync_copy(xs_hbm.at[src_v.at[j]], rows_v)
        pltpu.sync_copy(rows_v, acc_sh.at[dst_v.at[j]], add=True)
        return carry

    lax.fori_loop(0, NB, body, 0)
    plsc.subcore_barrier()
    pltpu.sync_copy(acc_sh.at[pl.ds(base, ROWS_PT)],
                    out_hbm.at[c, pl.ds(base, ROWS_PT)])


@functools.partial(
    pl.kernel, mesh=_mesh,
    out_type=jax.ShapeDtypeStruct((B, D), jnp.float32),
    scratch_types=[
        pltpu.VMEM((B // NW,), jnp.int32),
        pltpu.VMEM((B // NW, D), jnp.float32),
        pltpu.SemaphoreType.DMA,
    ],
)
def _sc_take(full_hbm, ptr_hbm, out_hbm, idx_v, rows_v, sem):
    c = lax.axis_index("c")
    s = lax.axis_index("s")
    wid = c * NS + s
    bpw = B // NW
    pltpu.sync_copy(ptr_hbm.at[pl.ds(wid * bpw, bpw)], idx_v)
    pltpu.async_copy(full_hbm.at[idx_v], rows_v, sem).wait()
    pltpu.sync_copy(rows_v, out_hbm.at[pl.ds(wid * bpw, bpw)])


def _dinv_of(degA_ref, degB_ref):
    deg = degA_ref[:, :1] + degB_ref[:, :1] + 1.0
    return lax.rsqrt(deg)


def _tc_pre_body(x_ref, w_ref, degA_ref, degB_ref, o_ref):
    dinv = _dinv_of(degA_ref, degB_ref)
    xw = jnp.dot(x_ref[...], w_ref[...], preferred_element_type=jnp.float32)
    o_ref[...] = xw * dinv


def _tc_mid_body(xs_ref, accA_ref, accB_ref, degA_ref, degB_ref, b_ref, w_ref, o_ref):
    dinv = _dinv_of(degA_ref, degB_ref)
    h = jax.nn.relu(dinv * (accA_ref[...] + accB_ref[...] + xs_ref[...]) + b_ref[...])
    o_ref[...] = jnp.dot(h, w_ref[...], preferred_element_type=jnp.float32) * dinv


def _tc_post_body(xs_ref, accA_ref, accB_ref, degA_ref, degB_ref, b_ref, o_ref):
    dinv = _dinv_of(degA_ref, degB_ref)
    o_ref[...] = dinv * (accA_ref[...] + accB_ref[...] + xs_ref[...]) + b_ref[...]


_tc_pre = pl.pallas_call(
    _tc_pre_body, out_shape=jax.ShapeDtypeStruct((N, D), jnp.float32))
_tc_mid = pl.pallas_call(
    _tc_mid_body, out_shape=jax.ShapeDtypeStruct((N, D), jnp.float32))
_tc_post = pl.pallas_call(
    _tc_post_body, out_shape=jax.ShapeDtypeStruct((N, D), jnp.float32))


def kernel(x, edge_index, ptr, W1, b1, W2, b2):
    src = edge_index[0].astype(jnp.int32)
    dst = edge_index[1].astype(jnp.int32)
    pad = E_PAD - E
    src3 = jnp.concatenate([src, jnp.zeros((pad,), jnp.int32)]).reshape(NW, NB, EBLK)
    dst3 = jnp.concatenate([dst, jnp.full((pad,), N, jnp.int32)]).reshape(NW, NB, EBLK)
    idxc = ptr[:-1].astype(jnp.int32)
    b1r = jnp.reshape(b1, (1, D))
    b2r = jnp.reshape(b2, (1, D))

    deg2 = _sc_deg(dst3)
    degA, degB = deg2[0, :N], deg2[1, :N]

    xs1 = _tc_pre(x, W1, degA, degB)
    acc1 = _sc_edge(xs1, src3, dst3)
    xs2 = _tc_mid(xs1, acc1[0, :N], acc1[1, :N], degA, degB, b1r, W2)
    acc2 = _sc_edge(xs2, src3, dst3)
    full = _tc_post(xs2, acc2[0, :N], acc2[1, :N], degA, degB, b2r)
    return _sc_take(full, idxc)


# speedup vs baseline: 20.2299x; 1.6470x over previous
import functools

import jax
import jax.numpy as jnp
from jax import lax
from jax.experimental import pallas as pl
from jax.experimental.pallas import tpu as pltpu
from jax.experimental.pallas import tpu_sc as plsc

N = 10000
E = 320000
D = 128
B = 256

NC = 2
NS = 16
NW = NC * NS

EBLK = 128
NB = (E + NW * EBLK - 1) // (NW * EBLK)
EPW = NB * EBLK
NSL = 10048
CAP = EPW + EBLK
CR = 512
CR_DUMMY = 256
E_PAD = NW * NB * EBLK
ROWS_PT = 640
NR = NS * ROWS_PT

_mesh = plsc.VectorSubcoreMesh(core_axis_name="c", subcore_axis_name="s")


def _zero_fill(zbuf, nrows, ncols):
    z16 = jnp.zeros((16,), jnp.float32)
    for r in range(nrows):
        for c in range(ncols // 16):
            zbuf[r, pl.ds(c * 16, 16)] = z16


@functools.partial(
    pl.kernel, mesh=_mesh,
    out_type=jax.ShapeDtypeStruct((NC, NR, 16), jnp.float32),
    scratch_types=[
        pltpu.VMEM((NB, EBLK), jnp.int32),
        pltpu.VMEM((EBLK, 16), jnp.float32),
        pltpu.VMEM((16, 16), jnp.float32),
        pltpu.VMEM_SHARED((NR, 16), jnp.float32),
    ],
)
def _sc_deg(dst_hbm, out_hbm, dst_v, ones_v, zbuf, deg_sh):
    c = lax.axis_index("c")
    s = lax.axis_index("s")
    wid = c * NS + s
    pltpu.sync_copy(dst_hbm.at[wid], dst_v)
    one16 = jnp.ones((16,), jnp.float32)
    for r in range(EBLK):
        ones_v[r] = one16
    _zero_fill(zbuf, 16, 16)
    base = s * ROWS_PT
    for b in range(ROWS_PT // 16):
        pltpu.sync_copy(zbuf, deg_sh.at[pl.ds(base + b * 16, 16)])
    plsc.subcore_barrier()

    def body(j, carry):
        pltpu.sync_copy(ones_v, deg_sh.at[dst_v.at[j]], add=True)
        return carry

    lax.fori_loop(0, NB, body, 0)
    plsc.subcore_barrier()
    pltpu.sync_copy(deg_sh.at[pl.ds(base, ROWS_PT)],
                    out_hbm.at[c, pl.ds(base, ROWS_PT)])


@functools.partial(
    pl.kernel, mesh=_mesh,
    out_type=jax.ShapeDtypeStruct((NC, NR, D), jnp.float32),
    scratch_types=[
        pltpu.VMEM((NB, EBLK), jnp.int32),
        pltpu.VMEM((NB, EBLK), jnp.int32),
        pltpu.VMEM((EBLK, D), jnp.float32),
        pltpu.VMEM((16, D), jnp.float32),
        pltpu.VMEM_SHARED((NR, D), jnp.float32),
    ],
)
def _sc_edge(xs_hbm, src_hbm, dst_hbm, out_hbm, src_v, dst_v, rows_v, zbuf, acc_sh):
    c = lax.axis_index("c")
    s = lax.axis_index("s")
    wid = c * NS + s
    pltpu.sync_copy(src_hbm.at[wid], src_v)
    pltpu.sync_copy(dst_hbm.at[wid], dst_v)
    _zero_fill(zbuf, 16, D)
    base = s * ROWS_PT
    for b in range(ROWS_PT // 16):
        pltpu.sync_copy(zbuf, acc_sh.at[pl.ds(base + b * 16, 16)])
    plsc.subcore_barrier()

    def body(j, carry):
        pltpu.sync_copy(xs_hbm.at[src_v.at[j]], rows_v)
        pltpu.sync_copy(rows_v, acc_sh.at[dst_v.at[j]], add=True)
        return carry

    lax.fori_loop(0, NB, body, 0)
    plsc.subcore_barrier()
    pltpu.sync_copy(acc_sh.at[pl.ds(base, ROWS_PT)],
                    out_hbm.at[c, pl.ds(base, ROWS_PT)])


@functools.partial(
    pl.kernel, mesh=_mesh,
    out_type=(jax.ShapeDtypeStruct((NC, CR, D), jnp.float32),
              jax.ShapeDtypeStruct((B,), jnp.int32)),
    scratch_types=[
        pltpu.VMEM((NSL,), jnp.int32),
        pltpu.VMEM((B,), jnp.int32),
        pltpu.VMEM((EPW,), jnp.int32),
        pltpu.VMEM((EPW,), jnp.int32),
        pltpu.VMEM((CAP,), jnp.int32),
        pltpu.VMEM((CAP // EBLK, EBLK), jnp.int32),
        pltpu.VMEM((EBLK, D), jnp.float32),
        pltpu.VMEM((B,), jnp.int32),
        pltpu.VMEM_SHARED((CR, D), jnp.float32),
    ],
    compiler_params=pltpu.CompilerParams(needs_layout_passes=False),
)
def _sc_edge2(xs_hbm, src_hbm, dst_hbm, ptr_hbm, zrows_hbm, acc_hbm, reps_hbm,
              slot_v, ptr_v, src_v, dst_v, csrc, cslot2, rows_v,
              reps_v, acc_sh):
    c = lax.axis_index("c")
    s = lax.axis_index("s")
    wid = c * NS + s
    pltpu.sync_copy(ptr_hbm, ptr_v)
    pltpu.sync_copy(src_hbm.at[wid], src_v)
    pltpu.sync_copy(dst_hbm.at[wid], dst_v)
    rpt = CR // NS
    pltpu.sync_copy(zrows_hbm.at[pl.ds(s * rpt, rpt)],
                    acc_sh.at[pl.ds(s * rpt, rpt)])
    neg1 = jnp.full((16,), -1, jnp.int32)
    for j in range(NSL // 16):
        slot_v[pl.ds(j * 16, 16)] = neg1
    lanes = lax.iota(jnp.int32, 16)
    for j in range(B // 16):
        pv = ptr_v[pl.ds(j * 16, 16)]
        plsc.store_scatter(slot_v, [pv], lanes + j * 16)
    def fbody(t, off):
        dv = dst_v[pl.ds(t * 16, 16)]
        g = plsc.load_gather(slot_v, [dv])
        m = g >= 0
        mi = m.astype(jnp.int32)
        cs = plsc.cumsum(mi)
        pos = off + cs - mi
        sv = src_v[pl.ds(t * 16, 16)]
        plsc.store_scatter(csrc, [pos], sv, mask=m)
        plsc.store_scatter(cslot2, [pos // EBLK, pos % EBLK], g, mask=m)
        return off + jnp.max(cs)

    off = lax.fori_loop(0, EPW // 16, fbody, 0)
    zero16 = jnp.zeros((16,), jnp.int32)
    dum16 = jnp.full((16,), CR_DUMMY, jnp.int32)
    for t in range(EBLK // 16):
        pp = off + lanes + t * 16
        plsc.store_scatter(csrc, [pp], zero16)
        plsc.store_scatter(cslot2, [pp // EBLK, pp % EBLK], dum16)
    nblk = (off + EBLK - 1) // EBLK
    plsc.subcore_barrier()

    def gbody(jb, carry):
        pltpu.sync_copy(xs_hbm.at[csrc.at[pl.ds(jb * EBLK, EBLK)]], rows_v)
        pltpu.sync_copy(rows_v, acc_sh.at[cslot2.at[jb]], add=True)
        return carry

    lax.fori_loop(0, nblk, gbody, 0)
    plsc.subcore_barrier()
    pltpu.sync_copy(acc_sh.at[pl.ds(s * rpt, rpt)],
                    acc_hbm.at[c, pl.ds(s * rpt, rpt)])

    @pl.when(jnp.logical_and(c == 0, s == 0))
    def _():
        for t in range(B // 16):
            pv = ptr_v[pl.ds(t * 16, 16)]
            reps_v[pl.ds(t * 16, 16)] = plsc.load_gather(slot_v, [pv])
        pltpu.sync_copy(reps_v, reps_hbm)


@functools.partial(
    pl.kernel, mesh=_mesh,
    out_type=jax.ShapeDtypeStruct((B, D), jnp.float32),
    scratch_types=[
        pltpu.VMEM((B // NW,), jnp.int32),
        pltpu.VMEM((B // NW,), jnp.int32),
        pltpu.VMEM((B // NW, D), jnp.float32),
        pltpu.VMEM((B // NW, D), jnp.float32),
        pltpu.VMEM((B // NW, D), jnp.float32),
        pltpu.VMEM((B // NW, D), jnp.float32),
        pltpu.VMEM((B // NW, D), jnp.float32),
        pltpu.VMEM((D,), jnp.float32),
        pltpu.SemaphoreType.DMA,
    ],
)
def _sc_take2(accA_hbm, accB_hbm, reps_hbm, xs_hbm, dv_hbm, ptr_hbm, b2_hbm,
              out_hbm, idx_v, rep_v, aA_v, aB_v, xsr_v, dvr_v, orow_v, b2_v, sem):
    c = lax.axis_index("c")
    s = lax.axis_index("s")
    wid = c * NS + s
    bpw = B // NW
    pltpu.sync_copy(ptr_hbm.at[pl.ds(wid * bpw, bpw)], idx_v)
    pltpu.sync_copy(reps_hbm.at[pl.ds(wid * bpw, bpw)], rep_v)
    pltpu.sync_copy(b2_hbm, b2_v)
    pltpu.async_copy(accA_hbm.at[rep_v], aA_v, sem).wait()
    pltpu.async_copy(accB_hbm.at[rep_v], aB_v, sem).wait()
    pltpu.async_copy(xs_hbm.at[idx_v], xsr_v, sem).wait()
    pltpu.async_copy(dv_hbm.at[idx_v], dvr_v, sem).wait()
    for i in range(bpw):
        for cc in range(D // 16):
            sl = pl.ds(cc * 16, 16)
            acc = aA_v[i, sl] + aB_v[i, sl] + xsr_v[i, sl]
            orow_v[i, sl] = acc * dvr_v[i, sl] + b2_v[sl]
    pltpu.sync_copy(orow_v, out_hbm.at[pl.ds(wid * bpw, bpw)])


def _dinv_of(degA_ref, degB_ref):
    deg = degA_ref[:, :1] + degB_ref[:, :1] + 1.0
    return lax.rsqrt(deg)


def _tc_pre_body(x_ref, w_ref, degA_ref, degB_ref, o_ref):
    dinv = _dinv_of(degA_ref, degB_ref)
    xw = jnp.dot(x_ref[...], w_ref[...], preferred_element_type=jnp.float32)
    o_ref[...] = xw * dinv


def _tc_mid_body(xs_ref, accA_ref, accB_ref, degA_ref, degB_ref, b_ref, w_ref,
                 o_ref, dv_ref):
    dinv = _dinv_of(degA_ref, degB_ref)
    h = jax.nn.relu(dinv * (accA_ref[...] + accB_ref[...] + xs_ref[...]) + b_ref[...])
    o_ref[...] = jnp.dot(h, w_ref[...], preferred_element_type=jnp.float32) * dinv
    dv_ref[...] = jnp.broadcast_to(dinv, (N, D))


_tc_pre = pl.pallas_call(
    _tc_pre_body, out_shape=jax.ShapeDtypeStruct((N, D), jnp.float32))
_tc_mid = pl.pallas_call(
    _tc_mid_body, out_shape=(jax.ShapeDtypeStruct((N, D), jnp.float32),
                             jax.ShapeDtypeStruct((N, D), jnp.float32)))


def kernel(x, edge_index, ptr, W1, b1, W2, b2):
    src = edge_index[0].astype(jnp.int32)
    dst = edge_index[1].astype(jnp.int32)
    pad = E_PAD - E
    src3 = jnp.concatenate([src, jnp.zeros((pad,), jnp.int32)]).reshape(NW, NB, EBLK)
    dst3 = jnp.concatenate([dst, jnp.full((pad,), N, jnp.int32)]).reshape(NW, NB, EBLK)
    src2 = src3.reshape(NW, EPW)
    dst2 = dst3.reshape(NW, EPW)
    idxc = ptr[:-1].astype(jnp.int32)
    b1r = jnp.reshape(b1, (1, D))

    deg2 = _sc_deg(dst3)
    degA, degB = deg2[0, :N], deg2[1, :N]

    xs1 = _tc_pre(x, W1, degA, degB)
    acc1 = _sc_edge(xs1, src3, dst3)
    xs2, dv16 = _tc_mid(xs1, acc1[0, :N], acc1[1, :N], degA, degB, b1r, W2)
    acc2, reps = _sc_edge2(xs2, src2, dst2, idxc,
                           jnp.zeros((CR, D), jnp.float32))
    return _sc_take2(acc2[0], acc2[1], reps, xs2, dv16, idxc,
                     b2.astype(jnp.float32))


# speedup vs baseline: 20.3187x; 1.0044x over previous
import functools

import jax
import jax.numpy as jnp
from jax import lax
from jax.experimental import pallas as pl
from jax.experimental.pallas import tpu as pltpu
from jax.experimental.pallas import tpu_sc as plsc

N = 10000
E = 320000
D = 128
B = 256

NC = 2
NS = 16
NW = NC * NS

EBLK = 128
NB = (E + NW * EBLK - 1) // (NW * EBLK)
EPW = NB * EBLK
NSL = 10240
CAP = EPW + EBLK
CR = 512
CR_DUMMY = 256
E_PAD = NW * NB * EBLK
ROWS_PT = 640
NR = NS * ROWS_PT

_mesh = plsc.VectorSubcoreMesh(core_axis_name="c", subcore_axis_name="s")


def _zero_fill(zbuf, nrows, ncols):
    z16 = jnp.zeros((16,), jnp.float32)
    for r in range(nrows):
        for c in range(ncols // 16):
            zbuf[r, pl.ds(c * 16, 16)] = z16


@functools.partial(
    pl.kernel, mesh=_mesh,
    out_type=jax.ShapeDtypeStruct((NC, NR, 16), jnp.float32),
    scratch_types=[
        pltpu.VMEM((NB, EBLK), jnp.int32),
        pltpu.VMEM((EBLK, 16), jnp.float32),
        pltpu.VMEM((16, 16), jnp.float32),
        pltpu.VMEM_SHARED((NR, 16), jnp.float32),
    ],
)
def _sc_deg(dst_hbm, out_hbm, dst_v, ones_v, zbuf, deg_sh):
    c = lax.axis_index("c")
    s = lax.axis_index("s")
    wid = c * NS + s
    pltpu.sync_copy(dst_hbm.at[wid], dst_v)
    one16 = jnp.ones((16,), jnp.float32)
    for r in range(EBLK):
        ones_v[r] = one16
    _zero_fill(zbuf, 16, 16)
    base = s * ROWS_PT
    for b in range(ROWS_PT // 16):
        pltpu.sync_copy(zbuf, deg_sh.at[pl.ds(base + b * 16, 16)])
    plsc.subcore_barrier()

    def body(j, carry):
        pltpu.sync_copy(ones_v, deg_sh.at[dst_v.at[j]], add=True)
        return carry

    lax.fori_loop(0, NB, body, 0)
    plsc.subcore_barrier()
    pltpu.sync_copy(deg_sh.at[pl.ds(base, ROWS_PT)],
                    out_hbm.at[c, pl.ds(base, ROWS_PT)])


@functools.partial(
    pl.kernel, mesh=_mesh,
    out_type=jax.ShapeDtypeStruct((NC, NR, D), jnp.float32),
    scratch_types=[
        pltpu.VMEM((NB, EBLK), jnp.int32),
        pltpu.VMEM((NB, EBLK), jnp.int32),
        pltpu.VMEM((EBLK, D), jnp.float32),
        pltpu.VMEM((16, D), jnp.float32),
        pltpu.VMEM_SHARED((NR, D), jnp.float32),
    ],
)
def _sc_edge(xs_hbm, src_hbm, dst_hbm, out_hbm, src_v, dst_v, rows_v, zbuf, acc_sh):
    c = lax.axis_index("c")
    s = lax.axis_index("s")
    wid = c * NS + s
    pltpu.sync_copy(src_hbm.at[wid], src_v)
    pltpu.sync_copy(dst_hbm.at[wid], dst_v)
    _zero_fill(zbuf, 16, D)
    base = s * ROWS_PT
    for b in range(ROWS_PT // 16):
        pltpu.sync_copy(zbuf, acc_sh.at[pl.ds(base + b * 16, 16)])
    plsc.subcore_barrier()

    def body(j, carry):
        pltpu.sync_copy(xs_hbm.at[src_v.at[j]], rows_v)
        pltpu.sync_copy(rows_v, acc_sh.at[dst_v.at[j]], add=True)
        return carry

    lax.fori_loop(0, NB, body, 0)
    plsc.subcore_barrier()
    pltpu.sync_copy(acc_sh.at[pl.ds(base, ROWS_PT)],
                    out_hbm.at[c, pl.ds(base, ROWS_PT)])


@functools.partial(
    pl.kernel, mesh=_mesh,
    out_type=(jax.ShapeDtypeStruct((NC, CR, D), jnp.float32),
              jax.ShapeDtypeStruct((B,), jnp.int32)),
    scratch_types=[
        pltpu.VMEM((NSL,), jnp.int32),
        pltpu.VMEM((B,), jnp.int32),
        pltpu.VMEM((EPW,), jnp.int32),
        pltpu.VMEM((EPW,), jnp.int32),
        pltpu.VMEM((CAP,), jnp.int32),
        pltpu.VMEM((CAP // EBLK, EBLK), jnp.int32),
        pltpu.VMEM((EBLK, D), jnp.float32),
        pltpu.VMEM((B,), jnp.int32),
        pltpu.VMEM_SHARED((CR, D), jnp.float32),
    ],
    compiler_params=pltpu.CompilerParams(needs_layout_passes=False),
)
def _sc_edge2(xs_hbm, src_hbm, dst_hbm, ptr_hbm, zrows_hbm, acc_hbm, reps_hbm,
              slot_v, ptr_v, src_v, dst_v, csrc, cslot2, rows_v,
              reps_v, acc_sh):
    c = lax.axis_index("c")
    s = lax.axis_index("s")
    wid = c * NS + s
    pltpu.sync_copy(ptr_hbm, ptr_v)
    pltpu.sync_copy(src_hbm.at[wid], src_v)
    pltpu.sync_copy(dst_hbm.at[wid], dst_v)
    rpt = CR // NS
    pltpu.sync_copy(zrows_hbm.at[pl.ds(s * rpt, rpt)],
                    acc_sh.at[pl.ds(s * rpt, rpt)])
    neg1 = jnp.full((16,), -1, jnp.int32)
    for j in range(NSL // 16):
        slot_v[pl.ds(j * 16, 16)] = neg1
    lanes = lax.iota(jnp.int32, 16)
    for j in range(B // 16):
        pv = ptr_v[pl.ds(j * 16, 16)]
        plsc.store_scatter(slot_v, [pv], lanes + j * 16)
    def fbody(t, off):
        dv = dst_v[pl.ds(t * 16, 16)]
        g = plsc.load_gather(slot_v, [dv])
        m = g >= 0
        mi = m.astype(jnp.int32)
        cs = plsc.cumsum(mi)
        pos = off + cs - mi
        sv = src_v[pl.ds(t * 16, 16)]
        plsc.store_scatter(csrc, [pos], sv, mask=m)
        plsc.store_scatter(cslot2, [pos // EBLK, pos % EBLK], g, mask=m)
        return off + jnp.max(cs)

    off = lax.fori_loop(0, EPW // 16, fbody, 0)
    zero16 = jnp.zeros((16,), jnp.int32)
    dum16 = jnp.full((16,), CR_DUMMY, jnp.int32)
    for t in range(EBLK // 16):
        pp = off + lanes + t * 16
        plsc.store_scatter(csrc, [pp], zero16)
        plsc.store_scatter(cslot2, [pp // EBLK, pp % EBLK], dum16)
    nblk = (off + EBLK - 1) // EBLK
    plsc.subcore_barrier()

    def gbody(jb, carry):
        pltpu.sync_copy(xs_hbm.at[csrc.at[pl.ds(jb * EBLK, EBLK)]], rows_v)
        pltpu.sync_copy(rows_v, acc_sh.at[cslot2.at[jb]], add=True)
        return carry

    lax.fori_loop(0, nblk, gbody, 0)
    plsc.subcore_barrier()
    pltpu.sync_copy(acc_sh.at[pl.ds(s * rpt, rpt)],
                    acc_hbm.at[c, pl.ds(s * rpt, rpt)])

    @pl.when(jnp.logical_and(c == 0, s == 0))
    def _():
        for t in range(B // 16):
            pv = ptr_v[pl.ds(t * 16, 16)]
            reps_v[pl.ds(t * 16, 16)] = plsc.load_gather(slot_v, [pv])
        pltpu.sync_copy(reps_v, reps_hbm)


@functools.partial(
    pl.kernel, mesh=_mesh,
    out_type=jax.ShapeDtypeStruct((B, D), jnp.float32),
    scratch_types=[
        pltpu.VMEM((B // NW,), jnp.int32),
        pltpu.VMEM((B // NW,), jnp.int32),
        pltpu.VMEM((B // NW, D), jnp.float32),
        pltpu.VMEM((B // NW, D), jnp.float32),
        pltpu.VMEM((B // NW, D), jnp.float32),
        pltpu.VMEM((B // NW, D), jnp.float32),
        pltpu.VMEM((B // NW, D), jnp.float32),
        pltpu.VMEM((D,), jnp.float32),
        pltpu.SemaphoreType.DMA,
    ],
)
def _sc_take2(accA_hbm, accB_hbm, reps_hbm, xs_hbm, dv_hbm, ptr_hbm, b2_hbm,
              out_hbm, idx_v, rep_v, aA_v, aB_v, xsr_v, dvr_v, orow_v, b2_v, sem):
    c = lax.axis_index("c")
    s = lax.axis_index("s")
    wid = c * NS + s
    bpw = B // NW
    pltpu.sync_copy(ptr_hbm.at[pl.ds(wid * bpw, bpw)], idx_v)
    pltpu.sync_copy(reps_hbm.at[pl.ds(wid * bpw, bpw)], rep_v)
    pltpu.sync_copy(b2_hbm, b2_v)
    pltpu.async_copy(accA_hbm.at[rep_v], aA_v, sem).wait()
    pltpu.async_copy(accB_hbm.at[rep_v], aB_v, sem).wait()
    pltpu.async_copy(xs_hbm.at[idx_v], xsr_v, sem).wait()
    pltpu.async_copy(dv_hbm.at[idx_v], dvr_v, sem).wait()
    for i in range(bpw):
        for cc in range(D // 16):
            sl = pl.ds(cc * 16, 16)
            acc = aA_v[i, sl] + aB_v[i, sl] + xsr_v[i, sl]
            orow_v[i, sl] = acc * dvr_v[i, sl] + b2_v[sl]
    pltpu.sync_copy(orow_v, out_hbm.at[pl.ds(wid * bpw, bpw)])


def _dinv_of(degA_ref, degB_ref):
    deg = degA_ref[:, :1] + degB_ref[:, :1] + 1.0
    return lax.rsqrt(deg)


def _tc_pre_body(x_ref, w_ref, degA_ref, degB_ref, o_ref):
    dinv = _dinv_of(degA_ref, degB_ref)
    xw = jnp.dot(x_ref[...], w_ref[...], preferred_element_type=jnp.float32)
    o_ref[...] = xw * dinv


def _tc_mid_body(xs_ref, accA_ref, accB_ref, degA_ref, degB_ref, b_ref, w_ref,
                 o_ref, dv_ref):
    dinv = _dinv_of(degA_ref, degB_ref)
    h = jax.nn.relu(dinv * (accA_ref[...] + accB_ref[...] + xs_ref[...]) + b_ref[...])
    o_ref[...] = jnp.dot(h, w_ref[...], preferred_element_type=jnp.float32) * dinv
    dv_ref[...] = jnp.broadcast_to(dinv, (N, D))


_tc_pre = pl.pallas_call(
    _tc_pre_body, out_shape=jax.ShapeDtypeStruct((N, D), jnp.float32))
_tc_mid = pl.pallas_call(
    _tc_mid_body, out_shape=(jax.ShapeDtypeStruct((N, D), jnp.float32),
                             jax.ShapeDtypeStruct((N, D), jnp.float32)))


def kernel(x, edge_index, ptr, W1, b1, W2, b2):
    src = edge_index[0].astype(jnp.int32)
    dst = edge_index[1].astype(jnp.int32)
    pad = E_PAD - E
    src3 = jnp.concatenate([src, jnp.zeros((pad,), jnp.int32)]).reshape(NW, NB, EBLK)
    dummy = N + (jnp.arange(pad, dtype=jnp.int32) % (NR - N))
    dst3 = jnp.concatenate([dst, dummy]).reshape(NW, NB, EBLK)
    src2 = src3.reshape(NW, EPW)
    dst2 = dst3.reshape(NW, EPW)
    idxc = ptr[:-1].astype(jnp.int32)
    b1r = jnp.reshape(b1, (1, D))

    deg2 = _sc_deg(dst3)
    degA, degB = deg2[0, :N], deg2[1, :N]

    xs1 = _tc_pre(x, W1, degA, degB)
    acc1 = _sc_edge(xs1, src3, dst3)
    xs2, dv16 = _tc_mid(xs1, acc1[0, :N], acc1[1, :N], degA, degB, b1r, W2)
    acc2, reps = _sc_edge2(xs2, src2, dst2, idxc,
                           jnp.zeros((CR, D), jnp.float32))
    return _sc_take2(acc2[0], acc2[1], reps, xs2, dv16, idxc,
                     b2.astype(jnp.float32))
